# Initial kernel scaffold; baseline (speedup 1.0000x reference)
import jax
import jax.numpy as jnp
from jax.experimental import pallas as pl


def kernel(ze, codebook, pos_emb, ln1_s, ln1_b, Wqkv, bqkv, Wo, bo, ln2_s, ln2_b, W1, b1, W2, b2, lnf_s, lnf_b, Wout, bout):
    raise NotImplementedError("write your pallas kernel here")


# speedup vs baseline: 1.7955x; 1.7955x over previous
import functools

import jax
import jax.numpy as jnp
from jax.experimental import pallas as pl

_D = 64
_DP = 588
_K = 1024
_S = 256
_B = 4
_H = 4
_NB = 2
_DFF = 256
_BS = _B * _S
_DH = _D // _H
_TQ = 128


def _layernorm(x, s, b):
    m = jnp.mean(x, axis=-1, keepdims=True)
    v = jnp.mean((x - m) * (x - m), axis=-1, keepdims=True)
    return (x - m) / jnp.sqrt(v + 1e-5) * s[None, :] + b[None, :]


def _gelu_tanh(x):
    c = 0.7978845608028654
    return 0.5 * x * (1.0 + jnp.tanh(c * (x + 0.044715 * (x * x * x))))


def _dot(a, b, dims):
    return jax.lax.dot_general(a, b, (dims, ((), ())),
                               precision=jax.lax.Precision.HIGHEST,
                               preferred_element_type=jnp.float32)


def vq_body(zef_ref, cb_ref, zq_ref, idx_ref):
    zef = zef_ref[...]
    cb = cb_ref[...]
    cn_col = jnp.sum(cb * cb, axis=1, keepdims=True)
    a_aug = jnp.concatenate(
        [zef * -2.0, jnp.ones((_TQ, 1), jnp.float32)], axis=1)
    b_aug = jnp.concatenate([cb, cn_col], axis=1)
    scores = _dot(a_aug, b_aug, (((1,), (1,))))
    mn = jnp.min(scores, axis=1, keepdims=True)
    kiota = jax.lax.broadcasted_iota(jnp.int32, (_TQ, _K), 1)
    idx = jnp.min(jnp.where(scores <= mn, kiota, _K), axis=1, keepdims=True)
    idx_ref[...] = idx
    onehot = (kiota == idx).astype(jnp.float32)
    zq_ref[...] = _dot(onehot, cb, (((1,), (0,))))


def vit_body(zq_ref, pos_ref, ln1s_ref, ln1b_ref, wqkv_ref, bqkv_ref,
             wo_ref, bo_ref, ln2s_ref, ln2b_ref, w1_ref, b1_ref, w2_ref,
             b2_ref, lnfs_ref, lnfb_ref, wout_ref, bout_ref, logits_ref):
    x = zq_ref[...] + pos_ref[...]
    head_of_lane = jax.lax.broadcasted_iota(jnp.int32, (_S, _D), 1) // _DH

    for i in range(_NB):
        h = _layernorm(x, ln1s_ref[i], ln1b_ref[i])
        qkv = _dot(h, wqkv_ref[i], (((1,), (0,)))) + bqkv_ref[i][None, :]
        q = qkv[:, 0:_D]
        k = qkv[:, _D:2 * _D]
        v = qkv[:, 2 * _D:3 * _D]
        qm = jnp.concatenate(
            [jnp.where(head_of_lane == hh, q, 0.0) for hh in range(_H)],
            axis=0)
        al = _dot(qm, k, (((1,), (1,)))) * 0.25
        al = al - jnp.max(al, axis=1, keepdims=True)
        e = jnp.exp(al)
        p = e / jnp.sum(e, axis=1, keepdims=True)
        ost = _dot(p, v, (((1,), (0,))))
        o = jnp.zeros((_S, _D), jnp.float32)
        for hh in range(_H):
            o = o + jnp.where(head_of_lane == hh,
                              ost[hh * _S:(hh + 1) * _S], 0.0)
        x = x + _dot(o, wo_ref[i], (((1,), (0,)))) + bo_ref[i][None, :]
        h2 = _layernorm(x, ln2s_ref[i], ln2b_ref[i])
        g = _dot(h2, w1_ref[i], (((1,), (0,)))) + b1_ref[i][None, :]
        x = x + _dot(_gelu_tanh(g), w2_ref[i], (((1,), (0,)))) \
            + b2_ref[i][None, :]

    xf = _layernorm(x, lnfs_ref[...], lnfb_ref[...])
    logits_ref[...] = _dot(xf, wout_ref[...], (((1,), (0,)))) \
        + bout_ref[...][None, :]


def _full(shape):
    return pl.BlockSpec(shape, lambda i: tuple(0 for _ in shape))


@functools.partial(jax.jit, static_argnames=("interpret",))
def _run(zef, codebook, pos_emb, ln1_s, ln1_b, Wqkv, bqkv, Wo, bo, ln2_s,
         ln2_b, W1, b1, W2, b2, lnf_s, lnf_b, Wout, bout, interpret=False):
    zq, idx = pl.pallas_call(
        vq_body,
        grid=(_BS // _TQ,),
        in_specs=[
            pl.BlockSpec((_TQ, _D), lambda i: (i, 0)),
            _full((_K, _D)),
        ],
        out_specs=(
            pl.BlockSpec((_TQ, _D), lambda i: (i, 0)),
            pl.BlockSpec((_TQ, 1), lambda i: (i, 0)),
        ),
        out_shape=(
            jax.ShapeDtypeStruct((_BS, _D), jnp.float32),
            jax.ShapeDtypeStruct((_BS, 1), jnp.int32),
        ),
        interpret=interpret,
    )(zef, codebook)

    logits = pl.pallas_call(
        vit_body,
        grid=(_B,),
        in_specs=[
            pl.BlockSpec((_S, _D), lambda i: (i, 0)),
            _full((_S, _D)),
            _full((_NB, _D)), _full((_NB, _D)),
            _full((_NB, _D, 3 * _D)), _full((_NB, 3 * _D)),
            _full((_NB, _D, _D)), _full((_NB, _D)),
            _full((_NB, _D)), _full((_NB, _D)),
            _full((_NB, _D, _DFF)), _full((_NB, _DFF)),
            _full((_NB, _DFF, _D)), _full((_NB, _D)),
            _full((_D,)), _full((_D,)),
            _full((_D, _DP)), _full((_DP,)),
        ],
        out_specs=pl.BlockSpec((_S, _DP), lambda i: (i, 0)),
        out_shape=jax.ShapeDtypeStruct((_BS, _DP), jnp.float32),
        interpret=interpret,
    )(zq, pos_emb, ln1_s, ln1_b, Wqkv, bqkv, Wo, bo, ln2_s, ln2_b,
      W1, b1, W2, b2, lnf_s, lnf_b, Wout, bout)
    return logits, zq


def kernel(ze, codebook, pos_emb, ln1_s, ln1_b, Wqkv, bqkv, Wo, bo, ln2_s,
           ln2_b, W1, b1, W2, b2, lnf_s, lnf_b, Wout, bout):
    zef = ze.reshape(_BS, _D)
    logits, zq = _run(zef, codebook, pos_emb, ln1_s, ln1_b, Wqkv, bqkv, Wo,
                      bo, ln2_s, ln2_b, W1, b1, W2, b2, lnf_s, lnf_b, Wout,
                      bout)
    return logits.reshape(_B, _S, _DP), zq.reshape(_B, _S, _D)


# speedup vs baseline: 3.1828x; 1.7727x over previous
import functools

import jax
import jax.numpy as jnp
from jax.experimental import pallas as pl
from jax.experimental.pallas import tpu as pltpu

_D = 64
_DP = 588
_K = 1024
_S = 256
_B = 4
_H = 4
_NB = 2
_DFF = 256
_BS = _B * _S
_DH = _D // _H
_TQ = 128
_NQ = _BS // _TQ


def _layernorm(x, s, b):
    m = jnp.mean(x, axis=-1, keepdims=True)
    v = jnp.mean((x - m) * (x - m), axis=-1, keepdims=True)
    return (x - m) / jnp.sqrt(v + 1e-5) * s[None, :] + b[None, :]


def _gelu_tanh(x):
    c = 0.7978845608028654
    return 0.5 * x * (1.0 + jnp.tanh(c * (x + 0.044715 * (x * x * x))))


def _dot(a, b, dims, prec=jax.lax.Precision.DEFAULT):
    return jax.lax.dot_general(a, b, (dims, ((), ())), precision=prec,
                               preferred_element_type=jnp.float32)


_HI = jax.lax.Precision.HIGHEST


def fused_body(zef_ref, cb_ref, pos_ref, ln1s_ref, ln1b_ref, wqkv_ref,
               bqkv_ref, wo_ref, bo_ref, ln2s_ref, ln2b_ref, w1_ref, b1_ref,
               w2_ref, b2_ref, lnfs_ref, lnfb_ref, wout_ref, bout_ref,
               logits_ref, zq_ref, zq_s):
    pid = pl.program_id(0)

    @pl.when(pid < _NQ)
    def _vq():
        zef = zef_ref[...]
        cb = cb_ref[...]
        cn_col = jnp.sum(cb * cb, axis=1, keepdims=True)
        a_aug = jnp.concatenate(
            [zef * -2.0, jnp.ones((_TQ, 1), jnp.float32)], axis=1)
        b_aug = jnp.concatenate([cb, cn_col], axis=1)
        scores = _dot(a_aug, b_aug, (((1,), (1,))), _HI)
        mn = jnp.min(scores, axis=1, keepdims=True)
        kiota = jax.lax.broadcasted_iota(jnp.int32, (_TQ, _K), 1)
        idx = jnp.min(jnp.where(scores <= mn, kiota, _K), axis=1,
                      keepdims=True)
        onehot = (kiota == idx).astype(jnp.float32)
        zq = _dot(onehot, cb, (((1,), (0,))), _HI)
        zq_ref[...] = zq
        zq_s[pl.ds(pid * _TQ, _TQ), :] = zq

    @pl.when(pid >= _NQ)
    def _vit():
        b = pid - _NQ
        x = zq_s[pl.ds(b * _S, _S), :] + pos_ref[...]
        head_of_lane = jax.lax.broadcasted_iota(jnp.int32, (_S, _D), 1) // _DH

        for i in range(_NB):
            h = _layernorm(x, ln1s_ref[i], ln1b_ref[i])
            qkv = _dot(h, wqkv_ref[i], (((1,), (0,)))) + bqkv_ref[i][None, :]
            q = qkv[:, 0:_D]
            k = qkv[:, _D:2 * _D]
            v = qkv[:, 2 * _D:3 * _D]
            qm = jnp.concatenate(
                [jnp.where(head_of_lane == hh, q, 0.0) for hh in range(_H)],
                axis=0)
            al = _dot(qm, k, (((1,), (1,)))) * 0.25
            al = al - jnp.max(al, axis=1, keepdims=True)
            e = jnp.exp(al)
            p = e / jnp.sum(e, axis=1, keepdims=True)
            ost = _dot(p, v, (((1,), (0,))))
            o = jnp.zeros((_S, _D), jnp.float32)
            for hh in range(_H):
                o = o + jnp.where(head_of_lane == hh,
                                  ost[hh * _S:(hh + 1) * _S], 0.0)
            x = x + _dot(o, wo_ref[i], (((1,), (0,)))) + bo_ref[i][None, :]
            h2 = _layernorm(x, ln2s_ref[i], ln2b_ref[i])
            g = _dot(h2, w1_ref[i], (((1,), (0,)))) + b1_ref[i][None, :]
            x = x + _dot(_gelu_tanh(g), w2_ref[i], (((1,), (0,)))) \
                + b2_ref[i][None, :]

        xf = _layernorm(x, lnfs_ref[...], lnfb_ref[...])
        logits_ref[...] = _dot(xf, wout_ref[...], (((1,), (0,)))) \
            + bout_ref[...][None, :]


def _full(shape):
    return pl.BlockSpec(shape, lambda i: tuple(0 for _ in shape))


@functools.partial(jax.jit, static_argnames=("interpret",))
def _run(zef, codebook, pos_emb, ln1_s, ln1_b, Wqkv, bqkv, Wo, bo, ln2_s,
         ln2_b, W1, b1, W2, b2, lnf_s, lnf_b, Wout, bout, interpret=False):
    logits, zq = pl.pallas_call(
        fused_body,
        grid=(_NQ + _B,),
        in_specs=[
            pl.BlockSpec((_TQ, _D), lambda i: (jnp.minimum(i, _NQ - 1), 0)),
            _full((_K, _D)),
            _full((_S, _D)),
            _full((_NB, _D)), _full((_NB, _D)),
            _full((_NB, _D, 3 * _D)), _full((_NB, 3 * _D)),
            _full((_NB, _D, _D)), _full((_NB, _D)),
            _full((_NB, _D)), _full((_NB, _D)),
            _full((_NB, _D, _DFF)), _full((_NB, _DFF)),
            _full((_NB, _DFF, _D)), _full((_NB, _D)),
            _full((_D,)), _full((_D,)),
            _full((_D, _DP)), _full((_DP,)),
        ],
        out_specs=(
            pl.BlockSpec((_S, _DP), lambda i: (jnp.maximum(i - _NQ, 0), 0)),
            pl.BlockSpec((_TQ, _D), lambda i: (jnp.minimum(i, _NQ - 1), 0)),
        ),
        out_shape=(
            jax.ShapeDtypeStruct((_BS, _DP), jnp.float32),
            jax.ShapeDtypeStruct((_BS, _D), jnp.float32),
        ),
        scratch_shapes=[pltpu.VMEM((_BS, _D), jnp.float32)],
        interpret=interpret,
    )(zef, codebook, pos_emb, ln1_s, ln1_b, Wqkv, bqkv, Wo, bo, ln2_s,
      ln2_b, W1, b1, W2, b2, lnf_s, lnf_b, Wout, bout)
    return logits, zq


def kernel(ze, codebook, pos_emb, ln1_s, ln1_b, Wqkv, bqkv, Wo, bo, ln2_s,
           ln2_b, W1, b1, W2, b2, lnf_s, lnf_b, Wout, bout):
    zef = ze.reshape(_BS, _D)
    logits, zq = _run(zef, codebook, pos_emb, ln1_s, ln1_b, Wqkv, bqkv, Wo,
                      bo, ln2_s, ln2_b, W1, b1, W2, b2, lnf_s, lnf_b, Wout,
                      bout)
    return logits.reshape(_B, _S, _DP), zq.reshape(_B, _S, _D)
